# baseline (speedup 1.0000x reference)
import functools

import jax
import jax.numpy as jnp
from jax import lax
from jax.experimental import pallas as pl
from jax.experimental.pallas import tpu as pltpu
from jax.experimental.pallas import tpu_sc as plsc

N = 10000
E = 320000
H = 128

NC = 2
NS = 16
NW = NC * NS
CHUNK = 64
NCHUNKS = E // CHUNK
SCHUNK = 64
SNCHUNKS = E // SCHUNK
SCHUNKS_PER_SC = SNCHUNKS // NC
RCHUNK = 40
NRCHUNKS = N // RCHUNK

def _pre_body(x_ref, g_ref, b_ref, wea_ref, web_ref, wn1a_ref, wn2a_ref,
              be_ref, bn1_ref, bn2_ref, t_ref, q_ref, s_ref):
    x = x_ref[...]
    mu = jnp.mean(x, axis=1, keepdims=True)
    var = jnp.mean((x - mu) ** 2, axis=1, keepdims=True)
    xn = (x - mu) * lax.rsqrt(var + 1e-5) * g_ref[...] + b_ref[...]
    p = jnp.dot(xn, wea_ref[...], preferred_element_type=jnp.float32) + be_ref[...]
    r = jnp.dot(xn, wn1a_ref[...], preferred_element_type=jnp.float32) + bn1_ref[...]
    t_ref[:, :H] = p.astype(jnp.bfloat16)
    t_ref[:, H:] = r.astype(jnp.bfloat16)
    q_ref[...] = jnp.dot(xn, web_ref[...], preferred_element_type=jnp.float32)
    s_ref[...] = jnp.dot(xn, wn2a_ref[...], preferred_element_type=jnp.float32) + bn2_ref[...]


def _precompute(x, g_n, b_n, wea, web, wn1a, wn2a, be, bn1, bn2):
    bn = 2000
    full = lambda shape: pl.BlockSpec(shape, lambda i: (0, 0))
    return pl.pallas_call(
        _pre_body,
        grid=(N // bn,),
        in_specs=[
            pl.BlockSpec((bn, H), lambda i: (i, 0)),
            full((1, H)), full((1, H)),
            full((H, H)), full((H, H)), full((H, H)), full((H, H)),
            full((1, H)), full((1, H)), full((1, H)),
        ],
        out_specs=[
            pl.BlockSpec((bn, 2 * H), lambda i: (i, 0)),
            pl.BlockSpec((bn, H), lambda i: (i, 0)),
            pl.BlockSpec((bn, H), lambda i: (i, 0)),
        ],
        out_shape=[
            jax.ShapeDtypeStruct((N, 2 * H), jnp.bfloat16),
            jax.ShapeDtypeStruct((N, H), jnp.float32),
            jax.ShapeDtypeStruct((N, H), jnp.float32),
        ],
    )(x, g_n, b_n, wea, web, wn1a, wn2a, be, bn1, bn2)


@functools.cache
def _gather_kernel():
    mesh = plsc.VectorSubcoreMesh(core_axis_name="c", subcore_axis_name="s")
    return functools.partial(
        pl.kernel,
        mesh=mesh,
        out_type=[
            jax.ShapeDtypeStruct((E, H), jnp.int32),
            jax.ShapeDtypeStruct((E, H), jnp.float32),
            jax.ShapeDtypeStruct((NC * N, H), jnp.float32),
        ],
        scratch_types=[
            pltpu.VMEM((2, CHUNK), jnp.int32),
            pltpu.VMEM((2, CHUNK), jnp.int32),
            pltpu.VMEM((2, CHUNK, H), jnp.int32),
            pltpu.VMEM((2, CHUNK, H), jnp.float32),
            pltpu.VMEM((CHUNK, H), jnp.float32),
            pltpu.VMEM((RCHUNK, H), jnp.float32),
            pltpu.VMEM_SHARED((N, H), jnp.float32),
        ] + [pltpu.SemaphoreType.DMA] * 8,
    )(_gather_body)


_GNITER = (NCHUNKS + NW - 1) // NW


def _gather_body(t_hbm, q_hbm, row_hbm, col_hbm, tg_hbm, qg_hbm, cnt_hbm,
                 row_v, col_v, t_v, q_v, ones_v, zrb_v, cnt_sh,
                 sr0, sr1, sg0, sg1, st0, st1, sq0, sq1):
    cid = lax.axis_index("c")
    sid = lax.axis_index("s")
    wid = sid * NC + cid
    s_idx = (sr0, sr1)
    s_gat = (sg0, sg1)
    s_stt = (st0, st1)
    s_stq = (sq0, sq1)

    def fill_ones(i, carry):
        ones_v[i // 8, pl.ds((i % 8) * 16, 16)] = jnp.full((16,), 1.0, jnp.float32)
        return carry
    lax.fori_loop(0, CHUNK * 8, fill_ones, 0)

    def fill_zero(i, carry):
        zrb_v[i // 8, pl.ds((i % 8) * 16, 16)] = jnp.zeros((16,), jnp.float32)
        return carry
    lax.fori_loop(0, RCHUNK * 8, fill_zero, 0)

    def zchunk(i, carry):
        rc = sid + i * NS

        @pl.when(rc < NRCHUNKS)
        def _():
            pltpu.sync_copy(zrb_v, cnt_sh.at[pl.ds(rc * RCHUNK, RCHUNK)])

        return carry

    lax.fori_loop(0, (NRCHUNKS + NS - 1) // NS, zchunk, 0)
    plsc.subcore_barrier()

    def start_idx(slot, b):
        base = (wid + slot * NW) * CHUNK
        pltpu.async_copy(row_hbm.at[pl.ds(base, CHUNK)], row_v.at[b], s_idx[b])
        pltpu.async_copy(col_hbm.at[pl.ds(base, CHUNK)], col_v.at[b], s_idx[b])

    def drain_idx(b):
        pltpu.make_async_copy(row_hbm.at[pl.ds(0, CHUNK)], row_v.at[b], s_idx[b]).wait()
        pltpu.make_async_copy(col_hbm.at[pl.ds(0, CHUNK)], col_v.at[b], s_idx[b]).wait()

    for b in range(2):
        @pl.when(wid + b * NW < NCHUNKS)
        def _(b=b):
            start_idx(b, b)

    def outer(j, carry):
        for b in range(2):
            i = j * 2 + b
            ci = wid + i * NW

            @pl.when(ci < NCHUNKS)
            def _(b=b, i=i):
                drain_idx(b)

                @pl.when(i >= 2)
                def _():
                    pltpu.make_async_copy(
                        t_v.at[b], tg_hbm.at[pl.ds(0, CHUNK)], s_stt[b]).wait()
                    pltpu.make_async_copy(
                        q_v.at[b], qg_hbm.at[pl.ds(0, CHUNK)], s_stq[b]).wait()

                pltpu.async_copy(t_hbm.at[row_v.at[b]], t_v.at[b], s_gat[b])
                pltpu.async_copy(q_hbm.at[col_v.at[b]], q_v.at[b], s_gat[b])

        for b in range(2):
            i = j * 2 + b
            ci = wid + i * NW

            @pl.when(ci < NCHUNKS)
            def _(b=b, i=i, ci=ci):
                base = ci * CHUNK
                pltpu.make_async_copy(t_hbm.at[row_v.at[b]], t_v.at[b], s_gat[b]).wait()
                pltpu.make_async_copy(q_hbm.at[col_v.at[b]], q_v.at[b], s_gat[b]).wait()
                pltpu.async_copy(t_v.at[b], tg_hbm.at[pl.ds(base, CHUNK)], s_stt[b])
                pltpu.async_copy(q_v.at[b], qg_hbm.at[pl.ds(base, CHUNK)], s_stq[b])
                pltpu.sync_copy(ones_v, cnt_sh.at[col_v.at[b]], add=True)

                @pl.when(wid + (i + 2) * NW < NCHUNKS)
                def _():
                    start_idx(i + 2, b)

        return carry

    lax.fori_loop(0, (_GNITER + 1) // 2, outer, 0)

    for i in range(_GNITER - 4, _GNITER):
        b = i % 2

        @pl.when((wid + i * NW < NCHUNKS)
                 & (wid + (i + 2) * NW >= NCHUNKS))
        def _(b=b):
            pltpu.make_async_copy(t_v.at[b], tg_hbm.at[pl.ds(0, CHUNK)], s_stt[b]).wait()
            pltpu.make_async_copy(q_v.at[b], qg_hbm.at[pl.ds(0, CHUNK)], s_stq[b]).wait()

    plsc.subcore_barrier()

    def wchunk(i, carry):
        rc = sid + i * NS

        @pl.when(rc < NRCHUNKS)
        def _():
            rbase = rc * RCHUNK
            pltpu.sync_copy(cnt_sh.at[pl.ds(rbase, RCHUNK)], zrb_v)
            pltpu.sync_copy(zrb_v, cnt_hbm.at[pl.ds(cid * N + rbase, RCHUNK)])

        return carry

    lax.fori_loop(0, (NRCHUNKS + NS - 1) // NS, wchunk, 0)


def _edge_body(tg_ref, qg_ref, ea_ref, ge_ref, be_ln_ref, wec_ref, wn1b_ref,
               wem_ref, bem_ref, eout_ref, m_ref):
    ea = ea_ref[...]
    mu = jnp.mean(ea, axis=1, keepdims=True)
    var = jnp.mean((ea - mu) ** 2, axis=1, keepdims=True)
    en = (ea - mu) * lax.rsqrt(var + 1e-5) * ge_ref[...] + be_ln_ref[...]
    e = jnp.maximum(
        tg_ref[:, :H].astype(jnp.float32) + qg_ref[...].astype(jnp.float32)
        + jnp.dot(en, wec_ref[...], preferred_element_type=jnp.float32), 0.0)
    m_ref[...] = jnp.maximum(
        tg_ref[:, H:].astype(jnp.float32)
        + jnp.dot(e, wn1b_ref[...], preferred_element_type=jnp.float32),
        0.0)
    se = e * jax.nn.sigmoid(e)
    eout_ref[...] = ea + jnp.dot(
        se, wem_ref[...], preferred_element_type=jnp.float32) + bem_ref[...]


def _edge_stage(tg, qg, edge_attr, g_e, b_e, wec, wn1b, wem, bem):
    be_ = 1280
    full = lambda shape: pl.BlockSpec(shape, lambda i: (0, 0))
    return pl.pallas_call(
        _edge_body,
        grid=(E // be_,),
        in_specs=[
            pl.BlockSpec((be_, 2 * H), lambda i: (i, 0)),
            pl.BlockSpec((be_, H), lambda i: (i, 0)),
            pl.BlockSpec((be_, H), lambda i: (i, 0)),
            full((1, H)), full((1, H)),
            full((H, H)), full((H, H)), full((H, H)),
            full((1, H)),
        ],
        out_specs=[
            pl.BlockSpec((be_, H), lambda i: (i, 0)),
            pl.BlockSpec((be_, H), lambda i: (i, 0)),
        ],
        out_shape=[
            jax.ShapeDtypeStruct((E, H), jnp.float32),
            jax.ShapeDtypeStruct((E, H), jnp.float32),
        ],
    )(tg, qg, edge_attr, g_e, b_e, wec, wn1b, wem, bem)


@functools.cache
def _scatter_kernel():
    mesh = plsc.VectorSubcoreMesh(core_axis_name="c", subcore_axis_name="s")
    return functools.partial(
        pl.kernel,
        mesh=mesh,
        out_type=jax.ShapeDtypeStruct((NC * N, H), jnp.float32),
        scratch_types=[
            pltpu.VMEM((SCHUNK,), jnp.int32),
            pltpu.VMEM((SCHUNK, H), jnp.float32),
            pltpu.VMEM((RCHUNK, H), jnp.float32),
            pltpu.VMEM_SHARED((N, H), jnp.float32),
        ],
    )(_scatter_body)


def _scatter_body(m_hbm, col_hbm, sums_hbm, col_v, m_v, zbuf_v, sums_sh):
    cid = lax.axis_index("c")
    sid = lax.axis_index("s")

    def zfill(i, carry):
        zbuf_v[i // 8, pl.ds((i % 8) * 16, 16)] = jnp.zeros((16,), jnp.float32)
        return carry
    lax.fori_loop(0, RCHUNK * 8, zfill, 0)

    def zchunk(i, carry):
        rc = sid + i * NS

        @pl.when(rc < NRCHUNKS)
        def _():
            pltpu.sync_copy(zbuf_v, sums_sh.at[pl.ds(rc * RCHUNK, RCHUNK)])

        return carry

    lax.fori_loop(0, (NRCHUNKS + NS - 1) // NS, zchunk, 0)
    plsc.subcore_barrier()

    def body(i, carry):
        chunk = cid * SCHUNKS_PER_SC + sid + i * NS

        @pl.when(sid + i * NS < SCHUNKS_PER_SC)
        def _():
            base = chunk * SCHUNK
            pltpu.sync_copy(col_hbm.at[pl.ds(base, SCHUNK)], col_v)
            pltpu.sync_copy(m_hbm.at[pl.ds(base, SCHUNK)], m_v)
            pltpu.sync_copy(m_v, sums_sh.at[col_v], add=True)

        return carry

    lax.fori_loop(0, (SCHUNKS_PER_SC + NS - 1) // NS, body, 0)
    plsc.subcore_barrier()

    def wchunk(i, carry):
        rc = sid + i * NS

        @pl.when(rc < NRCHUNKS)
        def _():
            rbase = rc * RCHUNK
            obase = cid * N + rbase
            pltpu.sync_copy(sums_sh.at[pl.ds(rbase, RCHUNK)], zbuf_v)
            pltpu.sync_copy(zbuf_v, sums_hbm.at[pl.ds(obase, RCHUNK)])

        return carry

    lax.fori_loop(0, (NRCHUNKS + NS - 1) // NS, wchunk, 0)


def _node_body(x_ref, s_ref, p0_ref, p1_ref, c0_ref, c1_ref, wn2b_ref,
               wnm_ref, bnm_ref, xout_ref):
    cnt = c0_ref[:, 0:1] + c1_ref[:, 0:1]

    agg = (p0_ref[...] + p1_ref[...]) / jnp.maximum(cnt, 1.0)
    h = jnp.maximum(
        s_ref[...] + jnp.dot(agg, wn2b_ref[...], preferred_element_type=jnp.float32),
        0.0)
    sh = h * jax.nn.sigmoid(h)
    xout_ref[...] = x_ref[...] + jnp.dot(
        sh, wnm_ref[...], preferred_element_type=jnp.float32) + bnm_ref[...]


def _node_stage(x, s, p0, p1, c0, c1, wn2b, wnm, bnm):
    bn = 2000
    full = lambda shape: pl.BlockSpec(shape, lambda i: (0, 0))
    return pl.pallas_call(
        _node_body,
        grid=(N // bn,),
        in_specs=[
            pl.BlockSpec((bn, H), lambda i: (i, 0)),
            pl.BlockSpec((bn, H), lambda i: (i, 0)),
            pl.BlockSpec((bn, H), lambda i: (i, 0)),
            pl.BlockSpec((bn, H), lambda i: (i, 0)),
            pl.BlockSpec((bn, H), lambda i: (i, 0)),
            pl.BlockSpec((bn, H), lambda i: (i, 0)),
            full((H, H)), full((H, H)), full((1, H)),
        ],
        out_specs=pl.BlockSpec((bn, H), lambda i: (i, 0)),
        out_shape=jax.ShapeDtypeStruct((N, H), jnp.float32),
    )(x, s, p0, p1, c0, c1, wn2b, wnm, bnm)


def kernel(x, edge_index, edge_attr, u, batch, g_n, b_n, g_e, b_e,
           We, be, Wn1, bn1, Wn2, bn2, Wnm, bnm, Wem, bem):
    row = edge_index[0]
    col = edge_index[1]
    r2 = lambda v: v.reshape(1, H)

    t, q, s = _precompute(
        x, r2(g_n), r2(b_n), We[:H], We[H:2 * H], Wn1[:H], Wn2[:H],
        r2(be), r2(bn1), r2(bn2))

    t32 = lax.bitcast_convert_type(t.reshape(N, H, 2), jnp.int32)
    tg32, qg, cntp = _gather_kernel()(t32, q, row, col)
    tg = lax.bitcast_convert_type(tg32, jnp.bfloat16).reshape(E, 2 * H)

    e_out, m = _edge_stage(
        tg, qg, edge_attr, r2(g_e), r2(b_e), We[2 * H:], Wn1[H:], Wem, r2(bem))

    sums = _scatter_kernel()(m, col)

    x_out = _node_stage(
        x, s, sums[:N], sums[N:], cntp[:N], cntp[N:], Wn2[H:], Wnm, r2(bnm))

    return (x_out, e_out)


# speedup vs baseline: 2.3836x; 2.3836x over previous
import functools

import jax
import jax.numpy as jnp
from jax import lax
from jax.experimental import pallas as pl
from jax.experimental.pallas import tpu as pltpu
from jax.experimental.pallas import tpu_sc as plsc

N = 10000
E = 320000
H = 128

NC = 2
NS = 16
NW = NC * NS
CHUNK = 64
NCHUNKS = E // CHUNK
SCHUNK = 64
SNCHUNKS = E // SCHUNK
SCHUNKS_PER_SC = SNCHUNKS // NC
RCHUNK = 40
NRCHUNKS = N // RCHUNK

def _pre_body(x_ref, g_ref, b_ref, wea_ref, web_ref, wn1a_ref, wn2a_ref,
              be_ref, bn1_ref, bn2_ref, t_ref, q_ref, s_ref):
    x = x_ref[...]
    mu = jnp.mean(x, axis=1, keepdims=True)
    var = jnp.mean((x - mu) ** 2, axis=1, keepdims=True)
    xn = (x - mu) * lax.rsqrt(var + 1e-5) * g_ref[...] + b_ref[...]
    p = jnp.dot(xn, wea_ref[...], preferred_element_type=jnp.float32) + be_ref[...]
    r = jnp.dot(xn, wn1a_ref[...], preferred_element_type=jnp.float32) + bn1_ref[...]
    pu = lax.bitcast_convert_type(p.astype(jnp.bfloat16), jnp.uint16).astype(jnp.uint32)
    ru = lax.bitcast_convert_type(r.astype(jnp.bfloat16), jnp.uint16).astype(jnp.uint32)
    t_ref[...] = lax.bitcast_convert_type((ru << 16) | pu, jnp.int32)
    q_ref[...] = jnp.dot(xn, web_ref[...], preferred_element_type=jnp.float32)
    s_ref[...] = jnp.dot(xn, wn2a_ref[...], preferred_element_type=jnp.float32) + bn2_ref[...]


def _precompute(x, g_n, b_n, wea, web, wn1a, wn2a, be, bn1, bn2):
    bn = 2000
    full = lambda shape: pl.BlockSpec(shape, lambda i: (0, 0))
    return pl.pallas_call(
        _pre_body,
        grid=(N // bn,),
        in_specs=[
            pl.BlockSpec((bn, H), lambda i: (i, 0)),
            full((1, H)), full((1, H)),
            full((H, H)), full((H, H)), full((H, H)), full((H, H)),
            full((1, H)), full((1, H)), full((1, H)),
        ],
        out_specs=[
            pl.BlockSpec((bn, H), lambda i: (i, 0)),
            pl.BlockSpec((bn, H), lambda i: (i, 0)),
            pl.BlockSpec((bn, H), lambda i: (i, 0)),
        ],
        out_shape=[
            jax.ShapeDtypeStruct((N, H), jnp.int32),
            jax.ShapeDtypeStruct((N, H), jnp.float32),
            jax.ShapeDtypeStruct((N, H), jnp.float32),
        ],
    )(x, g_n, b_n, wea, web, wn1a, wn2a, be, bn1, bn2)


@functools.cache
def _gather_kernel():
    mesh = plsc.VectorSubcoreMesh(core_axis_name="c", subcore_axis_name="s")
    return functools.partial(
        pl.kernel,
        mesh=mesh,
        out_type=[
            jax.ShapeDtypeStruct((E, H), jnp.int32),
            jax.ShapeDtypeStruct((E, H), jnp.float32),
            jax.ShapeDtypeStruct((NC * N, H), jnp.float32),
        ],
        scratch_types=[
            pltpu.VMEM((2, CHUNK), jnp.int32),
            pltpu.VMEM((2, CHUNK), jnp.int32),
            pltpu.VMEM((2, CHUNK, H), jnp.int32),
            pltpu.VMEM((2, CHUNK, H), jnp.float32),
            pltpu.VMEM((CHUNK, H), jnp.float32),
            pltpu.VMEM((RCHUNK, H), jnp.float32),
            pltpu.VMEM_SHARED((N, H), jnp.float32),
        ] + [pltpu.SemaphoreType.DMA] * 8,
    )(_gather_body)


_GNITER = (NCHUNKS + NW - 1) // NW


def _gather_body(t_hbm, q_hbm, row_hbm, col_hbm, tg_hbm, qg_hbm, cnt_hbm,
                 row_v, col_v, t_v, q_v, ones_v, zrb_v, cnt_sh,
                 sr0, sr1, sg0, sg1, st0, st1, sq0, sq1):
    cid = lax.axis_index("c")
    sid = lax.axis_index("s")
    wid = sid * NC + cid
    s_idx = (sr0, sr1)
    s_gat = (sg0, sg1)
    s_stt = (st0, st1)
    s_stq = (sq0, sq1)

    def fill_ones(i, carry):
        ones_v[i // 8, pl.ds((i % 8) * 16, 16)] = jnp.full((16,), 1.0, jnp.float32)
        return carry
    lax.fori_loop(0, CHUNK * 8, fill_ones, 0)

    def fill_zero(i, carry):
        zrb_v[i // 8, pl.ds((i % 8) * 16, 16)] = jnp.zeros((16,), jnp.float32)
        return carry
    lax.fori_loop(0, RCHUNK * 8, fill_zero, 0)

    def zchunk(i, carry):
        rc = sid + i * NS

        @pl.when(rc < NRCHUNKS)
        def _():
            pltpu.sync_copy(zrb_v, cnt_sh.at[pl.ds(rc * RCHUNK, RCHUNK)])

        return carry

    lax.fori_loop(0, (NRCHUNKS + NS - 1) // NS, zchunk, 0)
    plsc.subcore_barrier()

    def start_idx(slot, b):
        base = (wid + slot * NW) * CHUNK
        pltpu.async_copy(row_hbm.at[pl.ds(base, CHUNK)], row_v.at[b], s_idx[b])
        pltpu.async_copy(col_hbm.at[pl.ds(base, CHUNK)], col_v.at[b], s_idx[b])

    def drain_idx(b):
        pltpu.make_async_copy(row_hbm.at[pl.ds(0, CHUNK)], row_v.at[b], s_idx[b]).wait()
        pltpu.make_async_copy(col_hbm.at[pl.ds(0, CHUNK)], col_v.at[b], s_idx[b]).wait()

    for b in range(2):
        @pl.when(wid + b * NW < NCHUNKS)
        def _(b=b):
            start_idx(b, b)

    def outer(j, carry):
        for b in range(2):
            i = j * 2 + b
            ci = wid + i * NW

            @pl.when(ci < NCHUNKS)
            def _(b=b, i=i):
                drain_idx(b)

                @pl.when(i >= 2)
                def _():
                    pltpu.make_async_copy(
                        t_v.at[b], tg_hbm.at[pl.ds(0, CHUNK)], s_stt[b]).wait()
                    pltpu.make_async_copy(
                        q_v.at[b], qg_hbm.at[pl.ds(0, CHUNK)], s_stq[b]).wait()

                pltpu.async_copy(t_hbm.at[row_v.at[b]], t_v.at[b], s_gat[b])
                pltpu.async_copy(q_hbm.at[col_v.at[b]], q_v.at[b], s_gat[b])

        for b in range(2):
            i = j * 2 + b
            ci = wid + i * NW

            @pl.when(ci < NCHUNKS)
            def _(b=b, i=i, ci=ci):
                base = ci * CHUNK
                pltpu.make_async_copy(t_hbm.at[row_v.at[b]], t_v.at[b], s_gat[b]).wait()
                pltpu.make_async_copy(q_hbm.at[col_v.at[b]], q_v.at[b], s_gat[b]).wait()
                pltpu.async_copy(t_v.at[b], tg_hbm.at[pl.ds(base, CHUNK)], s_stt[b])
                pltpu.async_copy(q_v.at[b], qg_hbm.at[pl.ds(base, CHUNK)], s_stq[b])
                pltpu.sync_copy(ones_v, cnt_sh.at[col_v.at[b]], add=True)

                @pl.when(wid + (i + 2) * NW < NCHUNKS)
                def _():
                    start_idx(i + 2, b)

        return carry

    lax.fori_loop(0, (_GNITER + 1) // 2, outer, 0)

    for i in range(_GNITER - 4, _GNITER):
        b = i % 2

        @pl.when((wid + i * NW < NCHUNKS)
                 & (wid + (i + 2) * NW >= NCHUNKS))
        def _(b=b):
            pltpu.make_async_copy(t_v.at[b], tg_hbm.at[pl.ds(0, CHUNK)], s_stt[b]).wait()
            pltpu.make_async_copy(q_v.at[b], qg_hbm.at[pl.ds(0, CHUNK)], s_stq[b]).wait()

    plsc.subcore_barrier()

    def wchunk(i, carry):
        rc = sid + i * NS

        @pl.when(rc < NRCHUNKS)
        def _():
            rbase = rc * RCHUNK
            pltpu.sync_copy(cnt_sh.at[pl.ds(rbase, RCHUNK)], zrb_v)
            pltpu.sync_copy(zrb_v, cnt_hbm.at[pl.ds(cid * N + rbase, RCHUNK)])

        return carry

    lax.fori_loop(0, (NRCHUNKS + NS - 1) // NS, wchunk, 0)


def _edge_body(tg_ref, qg_ref, ea_ref, ge_ref, be_ln_ref, wec_ref, wn1b_ref,
               wem_ref, bem_ref, eout_ref, m_ref):
    ea = ea_ref[...]
    mu = jnp.mean(ea, axis=1, keepdims=True)
    var = jnp.mean((ea - mu) ** 2, axis=1, keepdims=True)
    en = (ea - mu) * lax.rsqrt(var + 1e-5) * ge_ref[...] + be_ln_ref[...]
    w = lax.bitcast_convert_type(tg_ref[...], jnp.uint32)
    pg = lax.bitcast_convert_type(w.astype(jnp.uint16), jnp.bfloat16)
    rg = lax.bitcast_convert_type((w >> 16).astype(jnp.uint16), jnp.bfloat16)
    e = jnp.maximum(
        pg.astype(jnp.float32) + qg_ref[...]
        + jnp.dot(en, wec_ref[...], preferred_element_type=jnp.float32), 0.0)
    m_ref[...] = jnp.maximum(
        rg.astype(jnp.float32)
        + jnp.dot(e, wn1b_ref[...], preferred_element_type=jnp.float32),
        0.0)
    se = e * jax.nn.sigmoid(e)
    eout_ref[...] = ea + jnp.dot(
        se, wem_ref[...], preferred_element_type=jnp.float32) + bem_ref[...]


def _edge_stage(tg, qg, edge_attr, g_e, b_e, wec, wn1b, wem, bem):
    be_ = 1280
    full = lambda shape: pl.BlockSpec(shape, lambda i: (0, 0))
    return pl.pallas_call(
        _edge_body,
        grid=(E // be_,),
        in_specs=[
            pl.BlockSpec((be_, H), lambda i: (i, 0)),
            pl.BlockSpec((be_, H), lambda i: (i, 0)),
            pl.BlockSpec((be_, H), lambda i: (i, 0)),
            full((1, H)), full((1, H)),
            full((H, H)), full((H, H)), full((H, H)),
            full((1, H)),
        ],
        out_specs=[
            pl.BlockSpec((be_, H), lambda i: (i, 0)),
            pl.BlockSpec((be_, H), lambda i: (i, 0)),
        ],
        out_shape=[
            jax.ShapeDtypeStruct((E, H), jnp.float32),
            jax.ShapeDtypeStruct((E, H), jnp.float32),
        ],
    )(tg, qg, edge_attr, g_e, b_e, wec, wn1b, wem, bem)


@functools.cache
def _scatter_kernel():
    mesh = plsc.VectorSubcoreMesh(core_axis_name="c", subcore_axis_name="s")
    return functools.partial(
        pl.kernel,
        mesh=mesh,
        out_type=jax.ShapeDtypeStruct((NC * N, H), jnp.float32),
        scratch_types=[
            pltpu.VMEM((SCHUNK,), jnp.int32),
            pltpu.VMEM((SCHUNK, H), jnp.float32),
            pltpu.VMEM((RCHUNK, H), jnp.float32),
            pltpu.VMEM_SHARED((N, H), jnp.float32),
        ],
    )(_scatter_body)


def _scatter_body(m_hbm, col_hbm, sums_hbm, col_v, m_v, zbuf_v, sums_sh):
    cid = lax.axis_index("c")
    sid = lax.axis_index("s")

    def zfill(i, carry):
        zbuf_v[i // 8, pl.ds((i % 8) * 16, 16)] = jnp.zeros((16,), jnp.float32)
        return carry
    lax.fori_loop(0, RCHUNK * 8, zfill, 0)

    def zchunk(i, carry):
        rc = sid + i * NS

        @pl.when(rc < NRCHUNKS)
        def _():
            pltpu.sync_copy(zbuf_v, sums_sh.at[pl.ds(rc * RCHUNK, RCHUNK)])

        return carry

    lax.fori_loop(0, (NRCHUNKS + NS - 1) // NS, zchunk, 0)
    plsc.subcore_barrier()

    def body(i, carry):
        chunk = cid * SCHUNKS_PER_SC + sid + i * NS

        @pl.when(sid + i * NS < SCHUNKS_PER_SC)
        def _():
            base = chunk * SCHUNK
            pltpu.sync_copy(col_hbm.at[pl.ds(base, SCHUNK)], col_v)
            pltpu.sync_copy(m_hbm.at[pl.ds(base, SCHUNK)], m_v)
            pltpu.sync_copy(m_v, sums_sh.at[col_v], add=True)

        return carry

    lax.fori_loop(0, (SCHUNKS_PER_SC + NS - 1) // NS, body, 0)
    plsc.subcore_barrier()

    def wchunk(i, carry):
        rc = sid + i * NS

        @pl.when(rc < NRCHUNKS)
        def _():
            rbase = rc * RCHUNK
            obase = cid * N + rbase
            pltpu.sync_copy(sums_sh.at[pl.ds(rbase, RCHUNK)], zbuf_v)
            pltpu.sync_copy(zbuf_v, sums_hbm.at[pl.ds(obase, RCHUNK)])

        return carry

    lax.fori_loop(0, (NRCHUNKS + NS - 1) // NS, wchunk, 0)


def _node_body(x_ref, s_ref, p0_ref, p1_ref, c0_ref, c1_ref, wn2b_ref,
               wnm_ref, bnm_ref, xout_ref):
    cnt = c0_ref[:, 0:1] + c1_ref[:, 0:1]

    agg = (p0_ref[...] + p1_ref[...]) / jnp.maximum(cnt, 1.0)
    h = jnp.maximum(
        s_ref[...] + jnp.dot(agg, wn2b_ref[...], preferred_element_type=jnp.float32),
        0.0)
    sh = h * jax.nn.sigmoid(h)
    xout_ref[...] = x_ref[...] + jnp.dot(
        sh, wnm_ref[...], preferred_element_type=jnp.float32) + bnm_ref[...]


def _node_stage(x, s, p0, p1, c0, c1, wn2b, wnm, bnm):
    bn = 2000
    full = lambda shape: pl.BlockSpec(shape, lambda i: (0, 0))
    return pl.pallas_call(
        _node_body,
        grid=(N // bn,),
        in_specs=[
            pl.BlockSpec((bn, H), lambda i: (i, 0)),
            pl.BlockSpec((bn, H), lambda i: (i, 0)),
            pl.BlockSpec((bn, H), lambda i: (i, 0)),
            pl.BlockSpec((bn, H), lambda i: (i, 0)),
            pl.BlockSpec((bn, H), lambda i: (i, 0)),
            pl.BlockSpec((bn, H), lambda i: (i, 0)),
            full((H, H)), full((H, H)), full((1, H)),
        ],
        out_specs=pl.BlockSpec((bn, H), lambda i: (i, 0)),
        out_shape=jax.ShapeDtypeStruct((N, H), jnp.float32),
    )(x, s, p0, p1, c0, c1, wn2b, wnm, bnm)


def kernel(x, edge_index, edge_attr, u, batch, g_n, b_n, g_e, b_e,
           We, be, Wn1, bn1, Wn2, bn2, Wnm, bnm, Wem, bem):
    row = edge_index[0]
    col = edge_index[1]
    r2 = lambda v: v.reshape(1, H)

    t, q, s = _precompute(
        x, r2(g_n), r2(b_n), We[:H], We[H:2 * H], Wn1[:H], Wn2[:H],
        r2(be), r2(bn1), r2(bn2))

    tg, qg, cntp = _gather_kernel()(t, q, row, col)

    e_out, m = _edge_stage(
        tg, qg, edge_attr, r2(g_e), r2(b_e), We[2 * H:], Wn1[H:], Wem, r2(bem))

    sums = _scatter_kernel()(m, col)

    x_out = _node_stage(
        x, s, sums[:N], sums[N:], cntp[:N], cntp[N:], Wn2[H:], Wnm, r2(bnm))

    return (x_out, e_out)


# speedup vs baseline: 2.8571x; 1.1986x over previous
import functools

import jax
import jax.numpy as jnp
from jax import lax
from jax.experimental import pallas as pl
from jax.experimental.pallas import tpu as pltpu
from jax.experimental.pallas import tpu_sc as plsc

N = 10000
E = 320000
H = 128

NC = 2
NS = 16
NW = NC * NS
CHUNK = 64
NCHUNKS = E // CHUNK
SCHUNK = 64
SNCHUNKS = E // SCHUNK
SCHUNKS_PER_SC = SNCHUNKS // NC
RCHUNK = 40
NRCHUNKS = N // RCHUNK

def _pre_body(x_ref, g_ref, b_ref, wea_ref, web_ref, wn1a_ref, wn2a_ref,
              be_ref, bn1_ref, bn2_ref, t_ref, q_ref, s_ref):
    x = x_ref[...]
    mu = jnp.mean(x, axis=1, keepdims=True)
    var = jnp.mean((x - mu) ** 2, axis=1, keepdims=True)
    xn = (x - mu) * lax.rsqrt(var + 1e-5) * g_ref[...] + b_ref[...]
    p = jnp.dot(xn, wea_ref[...], preferred_element_type=jnp.float32) + be_ref[...]
    r = jnp.dot(xn, wn1a_ref[...], preferred_element_type=jnp.float32) + bn1_ref[...]
    pu = lax.bitcast_convert_type(p.astype(jnp.bfloat16), jnp.uint16).astype(jnp.uint32)
    ru = lax.bitcast_convert_type(r.astype(jnp.bfloat16), jnp.uint16).astype(jnp.uint32)
    t_ref[...] = lax.bitcast_convert_type((ru << 16) | pu, jnp.int32)
    q_ref[...] = jnp.dot(xn, web_ref[...], preferred_element_type=jnp.float32)
    s_ref[...] = jnp.dot(xn, wn2a_ref[...], preferred_element_type=jnp.float32) + bn2_ref[...]


def _precompute(x, g_n, b_n, wea, web, wn1a, wn2a, be, bn1, bn2):
    bn = 2000
    full = lambda shape: pl.BlockSpec(shape, lambda i: (0, 0))
    return pl.pallas_call(
        _pre_body,
        grid=(N // bn,),
        in_specs=[
            pl.BlockSpec((bn, H), lambda i: (i, 0)),
            full((1, H)), full((1, H)),
            full((H, H)), full((H, H)), full((H, H)), full((H, H)),
            full((1, H)), full((1, H)), full((1, H)),
        ],
        out_specs=[
            pl.BlockSpec((bn, H), lambda i: (i, 0)),
            pl.BlockSpec((bn, H), lambda i: (i, 0)),
            pl.BlockSpec((bn, H), lambda i: (i, 0)),
        ],
        out_shape=[
            jax.ShapeDtypeStruct((N, H), jnp.int32),
            jax.ShapeDtypeStruct((N, H), jnp.float32),
            jax.ShapeDtypeStruct((N, H), jnp.float32),
        ],
    )(x, g_n, b_n, wea, web, wn1a, wn2a, be, bn1, bn2)


@functools.cache
def _gather_kernel():
    mesh = plsc.VectorSubcoreMesh(core_axis_name="c", subcore_axis_name="s")
    return functools.partial(
        pl.kernel,
        mesh=mesh,
        out_type=[
            jax.ShapeDtypeStruct((E, H), jnp.int32),
            jax.ShapeDtypeStruct((E, H), jnp.float32),
            jax.ShapeDtypeStruct((NC * N, H), jnp.float32),
        ],
        scratch_types=[
            pltpu.VMEM((2, CHUNK), jnp.int32),
            pltpu.VMEM((2, CHUNK), jnp.int32),
            pltpu.VMEM((2, CHUNK, H), jnp.int32),
            pltpu.VMEM((2, CHUNK, H), jnp.float32),
            pltpu.VMEM((CHUNK, H), jnp.float32),
            pltpu.VMEM((RCHUNK, H), jnp.float32),
            pltpu.VMEM_SHARED((N, H), jnp.float32),
        ] + [pltpu.SemaphoreType.DMA] * 8,
    )(_gather_body)


_GNITER = (NCHUNKS + NW - 1) // NW


def _gather_body(t_hbm, q_hbm, row_hbm, col_hbm, tg_hbm, qg_hbm, cnt_hbm,
                 row_v, col_v, t_v, q_v, ones_v, zrb_v, cnt_sh,
                 sr0, sr1, sg0, sg1, st0, st1, sq0, sq1):
    cid = lax.axis_index("c")
    sid = lax.axis_index("s")
    wid = sid * NC + cid
    s_idx = (sr0, sr1)
    s_gat = (sg0, sg1)
    s_stt = (st0, st1)
    s_stq = (sq0, sq1)

    def fill_ones(i, carry):
        ones_v[i // 8, pl.ds((i % 8) * 16, 16)] = jnp.full((16,), 1.0, jnp.float32)
        return carry
    lax.fori_loop(0, CHUNK * 8, fill_ones, 0)

    def fill_zero(i, carry):
        zrb_v[i // 8, pl.ds((i % 8) * 16, 16)] = jnp.zeros((16,), jnp.float32)
        return carry
    lax.fori_loop(0, RCHUNK * 8, fill_zero, 0)

    def zchunk(i, carry):
        rc = sid + i * NS

        @pl.when(rc < NRCHUNKS)
        def _():
            pltpu.sync_copy(zrb_v, cnt_sh.at[pl.ds(rc * RCHUNK, RCHUNK)])

        return carry

    lax.fori_loop(0, (NRCHUNKS + NS - 1) // NS, zchunk, 0)
    plsc.subcore_barrier()

    def start_idx(slot, b):
        base = (wid + slot * NW) * CHUNK
        pltpu.async_copy(row_hbm.at[pl.ds(base, CHUNK)], row_v.at[b], s_idx[b])
        pltpu.async_copy(col_hbm.at[pl.ds(base, CHUNK)], col_v.at[b], s_idx[b])

    def drain_idx(b):
        pltpu.make_async_copy(row_hbm.at[pl.ds(0, CHUNK)], row_v.at[b], s_idx[b]).wait()
        pltpu.make_async_copy(col_hbm.at[pl.ds(0, CHUNK)], col_v.at[b], s_idx[b]).wait()

    for b in range(2):
        @pl.when(wid + b * NW < NCHUNKS)
        def _(b=b):
            start_idx(b, b)

    def outer(j, carry):
        for b in range(2):
            i = j * 2 + b
            ci = wid + i * NW

            @pl.when(ci < NCHUNKS)
            def _(b=b, i=i):
                drain_idx(b)

                @pl.when(i >= 2)
                def _():
                    pltpu.make_async_copy(
                        t_v.at[b], tg_hbm.at[pl.ds(0, CHUNK)], s_stt[b]).wait()
                    pltpu.make_async_copy(
                        q_v.at[b], qg_hbm.at[pl.ds(0, CHUNK)], s_stq[b]).wait()

                pltpu.async_copy(t_hbm.at[row_v.at[b]], t_v.at[b], s_gat[b])
                pltpu.async_copy(q_hbm.at[col_v.at[b]], q_v.at[b], s_gat[b])

        for b in range(2):
            i = j * 2 + b
            ci = wid + i * NW

            @pl.when(ci < NCHUNKS)
            def _(b=b, i=i, ci=ci):
                base = ci * CHUNK
                pltpu.make_async_copy(t_hbm.at[row_v.at[b]], t_v.at[b], s_gat[b]).wait()
                pltpu.make_async_copy(q_hbm.at[col_v.at[b]], q_v.at[b], s_gat[b]).wait()
                pltpu.async_copy(t_v.at[b], tg_hbm.at[pl.ds(base, CHUNK)], s_stt[b])
                pltpu.async_copy(q_v.at[b], qg_hbm.at[pl.ds(base, CHUNK)], s_stq[b])
                pltpu.sync_copy(ones_v, cnt_sh.at[col_v.at[b]], add=True)

                @pl.when(wid + (i + 2) * NW < NCHUNKS)
                def _():
                    start_idx(i + 2, b)

        return carry

    lax.fori_loop(0, (_GNITER + 1) // 2, outer, 0)

    for i in range(_GNITER - 4, _GNITER):
        b = i % 2

        @pl.when((wid + i * NW < NCHUNKS)
                 & (wid + (i + 2) * NW >= NCHUNKS))
        def _(b=b):
            pltpu.make_async_copy(t_v.at[b], tg_hbm.at[pl.ds(0, CHUNK)], s_stt[b]).wait()
            pltpu.make_async_copy(q_v.at[b], qg_hbm.at[pl.ds(0, CHUNK)], s_stq[b]).wait()

    plsc.subcore_barrier()

    def wchunk(i, carry):
        rc = sid + i * NS

        @pl.when(rc < NRCHUNKS)
        def _():
            rbase = rc * RCHUNK
            pltpu.sync_copy(cnt_sh.at[pl.ds(rbase, RCHUNK)], zrb_v)
            pltpu.sync_copy(zrb_v, cnt_hbm.at[pl.ds(cid * N + rbase, RCHUNK)])

        return carry

    lax.fori_loop(0, (NRCHUNKS + NS - 1) // NS, wchunk, 0)


def _edge_body(tg_ref, qg_ref, ea_ref, ge_ref, be_ln_ref, wec_ref, wn1b_ref,
               wem_ref, bem_ref, eout_ref, m_ref):
    ea = ea_ref[...]
    mu = jnp.mean(ea, axis=1, keepdims=True)
    var = jnp.mean((ea - mu) ** 2, axis=1, keepdims=True)
    en = (ea - mu) * lax.rsqrt(var + 1e-5) * ge_ref[...] + be_ln_ref[...]
    w = lax.bitcast_convert_type(tg_ref[...], jnp.uint32)
    pg = lax.bitcast_convert_type(w.astype(jnp.uint16), jnp.bfloat16)
    rg = lax.bitcast_convert_type((w >> 16).astype(jnp.uint16), jnp.bfloat16)
    e = jnp.maximum(
        pg.astype(jnp.float32) + qg_ref[...]
        + jnp.dot(en, wec_ref[...], preferred_element_type=jnp.float32), 0.0)
    m_ref[...] = jnp.maximum(
        rg.astype(jnp.float32)
        + jnp.dot(e, wn1b_ref[...], preferred_element_type=jnp.float32),
        0.0)
    se = e * jax.nn.sigmoid(e)
    eout_ref[...] = ea + jnp.dot(
        se, wem_ref[...], preferred_element_type=jnp.float32) + bem_ref[...]


def _edge_stage(tg, qg, edge_attr, g_e, b_e, wec, wn1b, wem, bem):
    be_ = 1280
    full = lambda shape: pl.BlockSpec(shape, lambda i: (0, 0))
    return pl.pallas_call(
        _edge_body,
        grid=(E // be_,),
        in_specs=[
            pl.BlockSpec((be_, H), lambda i: (i, 0)),
            pl.BlockSpec((be_, H), lambda i: (i, 0)),
            pl.BlockSpec((be_, H), lambda i: (i, 0)),
            full((1, H)), full((1, H)),
            full((H, H)), full((H, H)), full((H, H)),
            full((1, H)),
        ],
        out_specs=[
            pl.BlockSpec((be_, H), lambda i: (i, 0)),
            pl.BlockSpec((be_, H), lambda i: (i, 0)),
        ],
        out_shape=[
            jax.ShapeDtypeStruct((E, H), jnp.float32),
            jax.ShapeDtypeStruct((E, H), jnp.float32),
        ],
    )(tg, qg, edge_attr, g_e, b_e, wec, wn1b, wem, bem)


@functools.cache
def _scatter_kernel():
    mesh = plsc.VectorSubcoreMesh(core_axis_name="c", subcore_axis_name="s")
    return functools.partial(
        pl.kernel,
        mesh=mesh,
        out_type=jax.ShapeDtypeStruct((NC * N, H), jnp.float32),
        scratch_types=[
            pltpu.VMEM((2, SCHUNK), jnp.int32),
            pltpu.VMEM((2, SCHUNK, H), jnp.float32),
            pltpu.VMEM((RCHUNK, H), jnp.float32),
            pltpu.VMEM_SHARED((N, H), jnp.float32),
        ] + [pltpu.SemaphoreType.DMA] * 4,
    )(_scatter_body)


_SNITER = (SCHUNKS_PER_SC + NS - 1) // NS


def _scatter_body(m_hbm, col_hbm, sums_hbm, col_v, m_v, zbuf_v, sums_sh,
                  sc0, sc1, sm0, sm1):
    cid = lax.axis_index("c")
    sid = lax.axis_index("s")
    s_col = (sc0, sc1)
    s_m = (sm0, sm1)

    def zfill(i, carry):
        zbuf_v[i // 8, pl.ds((i % 8) * 16, 16)] = jnp.zeros((16,), jnp.float32)
        return carry
    lax.fori_loop(0, RCHUNK * 8, zfill, 0)

    def zchunk(i, carry):
        rc = sid + i * NS

        @pl.when(rc < NRCHUNKS)
        def _():
            pltpu.sync_copy(zbuf_v, sums_sh.at[pl.ds(rc * RCHUNK, RCHUNK)])

        return carry

    lax.fori_loop(0, (NRCHUNKS + NS - 1) // NS, zchunk, 0)
    plsc.subcore_barrier()

    def start_loads(slot, b):
        base = (cid * SCHUNKS_PER_SC + sid + slot * NS) * SCHUNK
        pltpu.async_copy(col_hbm.at[pl.ds(base, SCHUNK)], col_v.at[b], s_col[b])
        pltpu.async_copy(m_hbm.at[pl.ds(base, SCHUNK)], m_v.at[b], s_m[b])

    for b in range(2):
        @pl.when(sid + b * NS < SCHUNKS_PER_SC)
        def _(b=b):
            start_loads(b, b)

    def body(j, carry):
        for b in range(2):
            i = j * 2 + b

            @pl.when(sid + i * NS < SCHUNKS_PER_SC)
            def _(b=b, i=i):
                pltpu.make_async_copy(
                    col_hbm.at[pl.ds(0, SCHUNK)], col_v.at[b], s_col[b]).wait()
                pltpu.make_async_copy(
                    m_hbm.at[pl.ds(0, SCHUNK)], m_v.at[b], s_m[b]).wait()
                pltpu.sync_copy(m_v.at[b], sums_sh.at[col_v.at[b]], add=True)

                @pl.when(sid + (i + 2) * NS < SCHUNKS_PER_SC)
                def _():
                    start_loads(i + 2, b)

        return carry

    lax.fori_loop(0, (_SNITER + 1) // 2, body, 0)
    plsc.subcore_barrier()

    def wchunk(i, carry):
        rc = sid + i * NS

        @pl.when(rc < NRCHUNKS)
        def _():
            rbase = rc * RCHUNK
            obase = cid * N + rbase
            pltpu.sync_copy(sums_sh.at[pl.ds(rbase, RCHUNK)], zbuf_v)
            pltpu.sync_copy(zbuf_v, sums_hbm.at[pl.ds(obase, RCHUNK)])

        return carry

    lax.fori_loop(0, (NRCHUNKS + NS - 1) // NS, wchunk, 0)


def _node_body(x_ref, s_ref, p0_ref, p1_ref, c0_ref, c1_ref, wn2b_ref,
               wnm_ref, bnm_ref, xout_ref):
    cnt = c0_ref[:, 0:1] + c1_ref[:, 0:1]

    agg = (p0_ref[...] + p1_ref[...]) / jnp.maximum(cnt, 1.0)
    h = jnp.maximum(
        s_ref[...] + jnp.dot(agg, wn2b_ref[...], preferred_element_type=jnp.float32),
        0.0)
    sh = h * jax.nn.sigmoid(h)
    xout_ref[...] = x_ref[...] + jnp.dot(
        sh, wnm_ref[...], preferred_element_type=jnp.float32) + bnm_ref[...]


def _node_stage(x, s, p0, p1, c0, c1, wn2b, wnm, bnm):
    bn = 2000
    full = lambda shape: pl.BlockSpec(shape, lambda i: (0, 0))
    return pl.pallas_call(
        _node_body,
        grid=(N // bn,),
        in_specs=[
            pl.BlockSpec((bn, H), lambda i: (i, 0)),
            pl.BlockSpec((bn, H), lambda i: (i, 0)),
            pl.BlockSpec((bn, H), lambda i: (i, 0)),
            pl.BlockSpec((bn, H), lambda i: (i, 0)),
            pl.BlockSpec((bn, H), lambda i: (i, 0)),
            pl.BlockSpec((bn, H), lambda i: (i, 0)),
            full((H, H)), full((H, H)), full((1, H)),
        ],
        out_specs=pl.BlockSpec((bn, H), lambda i: (i, 0)),
        out_shape=jax.ShapeDtypeStruct((N, H), jnp.float32),
    )(x, s, p0, p1, c0, c1, wn2b, wnm, bnm)


def kernel(x, edge_index, edge_attr, u, batch, g_n, b_n, g_e, b_e,
           We, be, Wn1, bn1, Wn2, bn2, Wnm, bnm, Wem, bem):
    row = edge_index[0]
    col = edge_index[1]
    r2 = lambda v: v.reshape(1, H)

    t, q, s = _precompute(
        x, r2(g_n), r2(b_n), We[:H], We[H:2 * H], Wn1[:H], Wn2[:H],
        r2(be), r2(bn1), r2(bn2))

    tg, qg, cntp = _gather_kernel()(t, q, row, col)

    e_out, m = _edge_stage(
        tg, qg, edge_attr, r2(g_e), r2(b_e), We[2 * H:], Wn1[H:], Wem, r2(bem))

    sums = _scatter_kernel()(m, col)

    x_out = _node_stage(
        x, s, sums[:N], sums[N:], cntp[:N], cntp[N:], Wn2[H:], Wnm, r2(bnm))

    return (x_out, e_out)


# speedup vs baseline: 3.0711x; 1.0749x over previous
import functools

import jax
import jax.numpy as jnp
from jax import lax
from jax.experimental import pallas as pl
from jax.experimental.pallas import tpu as pltpu
from jax.experimental.pallas import tpu_sc as plsc

N = 10000
E = 320000
H = 128

NC = 2
NS = 16
NW = NC * NS
EH = E // 2
CHUNK = 64
NCHUNKS = EH // CHUNK
SCHUNK = 64
SNCHUNKS = EH // SCHUNK
SCHUNKS_PER_SC = SNCHUNKS // NC
RCHUNK = 40
NRCHUNKS = N // RCHUNK

def _pre_body(x_ref, g_ref, b_ref, wea_ref, web_ref, wn1a_ref, wn2a_ref,
              be_ref, bn1_ref, bn2_ref, t_ref, q_ref, s_ref):
    x = x_ref[...]
    mu = jnp.mean(x, axis=1, keepdims=True)
    var = jnp.mean((x - mu) ** 2, axis=1, keepdims=True)
    xn = (x - mu) * lax.rsqrt(var + 1e-5) * g_ref[...] + b_ref[...]
    p = jnp.dot(xn, wea_ref[...], preferred_element_type=jnp.float32) + be_ref[...]
    r = jnp.dot(xn, wn1a_ref[...], preferred_element_type=jnp.float32) + bn1_ref[...]
    pu = lax.bitcast_convert_type(p.astype(jnp.bfloat16), jnp.uint16).astype(jnp.uint32)
    ru = lax.bitcast_convert_type(r.astype(jnp.bfloat16), jnp.uint16).astype(jnp.uint32)
    t_ref[...] = lax.bitcast_convert_type((ru << 16) | pu, jnp.int32)
    q_ref[...] = jnp.dot(xn, web_ref[...], preferred_element_type=jnp.float32)
    s_ref[...] = jnp.dot(xn, wn2a_ref[...], preferred_element_type=jnp.float32) + bn2_ref[...]


def _precompute(x, g_n, b_n, wea, web, wn1a, wn2a, be, bn1, bn2):
    bn = 2000
    full = lambda shape: pl.BlockSpec(shape, lambda i: (0, 0))
    return pl.pallas_call(
        _pre_body,
        grid=(N // bn,),
        in_specs=[
            pl.BlockSpec((bn, H), lambda i: (i, 0)),
            full((1, H)), full((1, H)),
            full((H, H)), full((H, H)), full((H, H)), full((H, H)),
            full((1, H)), full((1, H)), full((1, H)),
        ],
        out_specs=[
            pl.BlockSpec((bn, H), lambda i: (i, 0)),
            pl.BlockSpec((bn, H), lambda i: (i, 0)),
            pl.BlockSpec((bn, H), lambda i: (i, 0)),
        ],
        out_shape=[
            jax.ShapeDtypeStruct((N, H), jnp.int32),
            jax.ShapeDtypeStruct((N, H), jnp.float32),
            jax.ShapeDtypeStruct((N, H), jnp.float32),
        ],
    )(x, g_n, b_n, wea, web, wn1a, wn2a, be, bn1, bn2)


@functools.cache
def _gather_kernel(half):
    mesh = plsc.VectorSubcoreMesh(core_axis_name="c", subcore_axis_name="s")
    return functools.partial(
        pl.kernel,
        mesh=mesh,
        out_type=[
            jax.ShapeDtypeStruct((EH, H), jnp.int32),
            jax.ShapeDtypeStruct((EH, H), jnp.float32),
            jax.ShapeDtypeStruct((NC * N, H), jnp.float32),
        ],
        scratch_types=[
            pltpu.VMEM((2, CHUNK), jnp.int32),
            pltpu.VMEM((2, CHUNK), jnp.int32),
            pltpu.VMEM((2, CHUNK, H), jnp.int32),
            pltpu.VMEM((2, CHUNK, H), jnp.float32),
            pltpu.VMEM((CHUNK, H), jnp.float32),
            pltpu.VMEM((RCHUNK, H), jnp.float32),
            pltpu.VMEM_SHARED((N, H), jnp.float32),
        ] + [pltpu.SemaphoreType.DMA] * 8,
    )(functools.partial(_gather_body, half))


_GNITER = (NCHUNKS + NW - 1) // NW


def _gather_body(half, t_hbm, q_hbm, row_hbm, col_hbm, tg_hbm, qg_hbm, cnt_hbm,
                 row_v, col_v, t_v, q_v, ones_v, zrb_v, cnt_sh,
                 sr0, sr1, sg0, sg1, st0, st1, sq0, sq1):
    goff = half * EH
    cid = lax.axis_index("c")
    sid = lax.axis_index("s")
    wid = sid * NC + cid
    s_idx = (sr0, sr1)
    s_gat = (sg0, sg1)
    s_stt = (st0, st1)
    s_stq = (sq0, sq1)

    def fill_ones(i, carry):
        ones_v[i // 8, pl.ds((i % 8) * 16, 16)] = jnp.full((16,), 1.0, jnp.float32)
        return carry
    lax.fori_loop(0, CHUNK * 8, fill_ones, 0)

    def fill_zero(i, carry):
        zrb_v[i // 8, pl.ds((i % 8) * 16, 16)] = jnp.zeros((16,), jnp.float32)
        return carry
    lax.fori_loop(0, RCHUNK * 8, fill_zero, 0)

    def zchunk(i, carry):
        rc = sid + i * NS

        @pl.when(rc < NRCHUNKS)
        def _():
            pltpu.sync_copy(zrb_v, cnt_sh.at[pl.ds(rc * RCHUNK, RCHUNK)])

        return carry

    lax.fori_loop(0, (NRCHUNKS + NS - 1) // NS, zchunk, 0)
    plsc.subcore_barrier()

    def start_idx(slot, b):
        base = goff + (wid + slot * NW) * CHUNK
        pltpu.async_copy(row_hbm.at[pl.ds(base, CHUNK)], row_v.at[b], s_idx[b])
        pltpu.async_copy(col_hbm.at[pl.ds(base, CHUNK)], col_v.at[b], s_idx[b])

    def drain_idx(b):
        pltpu.make_async_copy(row_hbm.at[pl.ds(0, CHUNK)], row_v.at[b], s_idx[b]).wait()
        pltpu.make_async_copy(col_hbm.at[pl.ds(0, CHUNK)], col_v.at[b], s_idx[b]).wait()

    for b in range(2):
        @pl.when(wid + b * NW < NCHUNKS)
        def _(b=b):
            start_idx(b, b)

    def outer(j, carry):
        for b in range(2):
            i = j * 2 + b
            ci = wid + i * NW

            @pl.when(ci < NCHUNKS)
            def _(b=b, i=i):
                drain_idx(b)

                @pl.when(i >= 2)
                def _():
                    pltpu.make_async_copy(
                        t_v.at[b], tg_hbm.at[pl.ds(0, CHUNK)], s_stt[b]).wait()
                    pltpu.make_async_copy(
                        q_v.at[b], qg_hbm.at[pl.ds(0, CHUNK)], s_stq[b]).wait()

                pltpu.async_copy(t_hbm.at[row_v.at[b]], t_v.at[b], s_gat[b])
                pltpu.async_copy(q_hbm.at[col_v.at[b]], q_v.at[b], s_gat[b])

        for b in range(2):
            i = j * 2 + b
            ci = wid + i * NW

            @pl.when(ci < NCHUNKS)
            def _(b=b, i=i, ci=ci):
                base = ci * CHUNK
                pltpu.make_async_copy(t_hbm.at[row_v.at[b]], t_v.at[b], s_gat[b]).wait()
                pltpu.make_async_copy(q_hbm.at[col_v.at[b]], q_v.at[b], s_gat[b]).wait()
                pltpu.async_copy(t_v.at[b], tg_hbm.at[pl.ds(base, CHUNK)], s_stt[b])
                pltpu.async_copy(q_v.at[b], qg_hbm.at[pl.ds(base, CHUNK)], s_stq[b])
                pltpu.sync_copy(ones_v, cnt_sh.at[col_v.at[b]], add=True)

                @pl.when(wid + (i + 2) * NW < NCHUNKS)
                def _():
                    start_idx(i + 2, b)

        return carry

    lax.fori_loop(0, (_GNITER + 1) // 2, outer, 0)

    for i in range(_GNITER - 4, _GNITER):
        b = i % 2

        @pl.when((wid + i * NW < NCHUNKS)
                 & (wid + (i + 2) * NW >= NCHUNKS))
        def _(b=b):
            pltpu.make_async_copy(t_v.at[b], tg_hbm.at[pl.ds(0, CHUNK)], s_stt[b]).wait()
            pltpu.make_async_copy(q_v.at[b], qg_hbm.at[pl.ds(0, CHUNK)], s_stq[b]).wait()

    plsc.subcore_barrier()

    def wchunk(i, carry):
        rc = sid + i * NS

        @pl.when(rc < NRCHUNKS)
        def _():
            rbase = rc * RCHUNK
            pltpu.sync_copy(cnt_sh.at[pl.ds(rbase, RCHUNK)], zrb_v)
            pltpu.sync_copy(zrb_v, cnt_hbm.at[pl.ds(cid * N + rbase, RCHUNK)])

        return carry

    lax.fori_loop(0, (NRCHUNKS + NS - 1) // NS, wchunk, 0)


def _edge_body(tg_ref, qg_ref, ea_ref, ge_ref, be_ln_ref, wec_ref, wn1b_ref,
               wem_ref, bem_ref, *refs):
    eout_ref, m_ref = refs[-2], refs[-1]
    ea = ea_ref[...]
    mu = jnp.mean(ea, axis=1, keepdims=True)
    var = jnp.mean((ea - mu) ** 2, axis=1, keepdims=True)
    en = (ea - mu) * lax.rsqrt(var + 1e-5) * ge_ref[...] + be_ln_ref[...]
    w = lax.bitcast_convert_type(tg_ref[...], jnp.uint32)
    pg = lax.bitcast_convert_type(w.astype(jnp.uint16), jnp.bfloat16)
    rg = lax.bitcast_convert_type((w >> 16).astype(jnp.uint16), jnp.bfloat16)
    e = jnp.maximum(
        pg.astype(jnp.float32) + qg_ref[...]
        + jnp.dot(en, wec_ref[...], preferred_element_type=jnp.float32), 0.0)
    m_ref[...] = jnp.maximum(
        rg.astype(jnp.float32)
        + jnp.dot(e, wn1b_ref[...], preferred_element_type=jnp.float32),
        0.0)
    se = e * jax.nn.sigmoid(e)
    eout_ref[...] = ea + jnp.dot(
        se, wem_ref[...], preferred_element_type=jnp.float32) + bem_ref[...]


def _edge_stage(half, eout_prev, tg, qg, edge_attr, g_e, b_e, wec, wn1b, wem, bem):
    be_ = 1280
    nsteps = EH // be_
    off = half * nsteps
    full = lambda shape: pl.BlockSpec(shape, lambda i: (0, 0))
    in_specs = [
        pl.BlockSpec((be_, H), lambda i: (i, 0)),
        pl.BlockSpec((be_, H), lambda i: (i, 0)),
        pl.BlockSpec((be_, H), lambda i: (i + off, 0)),
        full((1, H)), full((1, H)),
        full((H, H)), full((H, H)), full((H, H)),
        full((1, H)),
    ]
    args = [tg, qg, edge_attr, g_e, b_e, wec, wn1b, wem, bem]
    aliases = {}
    if half:
        in_specs.append(pl.BlockSpec((be_, H), lambda i: (i + off, 0)))
        args.append(eout_prev)
        aliases = {9: 0}
    return pl.pallas_call(
        _edge_body,
        grid=(nsteps,),
        in_specs=in_specs,
        out_specs=[
            pl.BlockSpec((be_, H), lambda i: (i + off, 0)),
            pl.BlockSpec((be_, H), lambda i: (i, 0)),
        ],
        out_shape=[
            jax.ShapeDtypeStruct((E, H), jnp.float32),
            jax.ShapeDtypeStruct((EH, H), jnp.float32),
        ],
        input_output_aliases=aliases,
    )(*args)


@functools.cache
def _scatter_kernel(half):
    mesh = plsc.VectorSubcoreMesh(core_axis_name="c", subcore_axis_name="s")
    return functools.partial(
        pl.kernel,
        mesh=mesh,
        out_type=jax.ShapeDtypeStruct((NC * N, H), jnp.float32),
        scratch_types=[
            pltpu.VMEM((2, SCHUNK), jnp.int32),
            pltpu.VMEM((2, SCHUNK, H), jnp.float32),
            pltpu.VMEM((RCHUNK, H), jnp.float32),
            pltpu.VMEM_SHARED((N, H), jnp.float32),
        ] + [pltpu.SemaphoreType.DMA] * 4,
    )(functools.partial(_scatter_body, half))


_SNITER = (SCHUNKS_PER_SC + NS - 1) // NS


def _scatter_body(half, m_hbm, col_hbm, sums_hbm, col_v, m_v, zbuf_v, sums_sh,
                  sc0, sc1, sm0, sm1):
    cid = lax.axis_index("c")
    sid = lax.axis_index("s")
    s_col = (sc0, sc1)
    s_m = (sm0, sm1)

    def zfill(i, carry):
        zbuf_v[i // 8, pl.ds((i % 8) * 16, 16)] = jnp.zeros((16,), jnp.float32)
        return carry
    lax.fori_loop(0, RCHUNK * 8, zfill, 0)

    def zchunk(i, carry):
        rc = sid + i * NS

        @pl.when(rc < NRCHUNKS)
        def _():
            pltpu.sync_copy(zbuf_v, sums_sh.at[pl.ds(rc * RCHUNK, RCHUNK)])

        return carry

    lax.fori_loop(0, (NRCHUNKS + NS - 1) // NS, zchunk, 0)
    plsc.subcore_barrier()

    def start_loads(slot, b):
        lbase = (cid * SCHUNKS_PER_SC + sid + slot * NS) * SCHUNK
        pltpu.async_copy(col_hbm.at[pl.ds(half * EH + lbase, SCHUNK)],
                         col_v.at[b], s_col[b])
        pltpu.async_copy(m_hbm.at[pl.ds(lbase, SCHUNK)], m_v.at[b], s_m[b])

    for b in range(2):
        @pl.when(sid + b * NS < SCHUNKS_PER_SC)
        def _(b=b):
            start_loads(b, b)

    def body(j, carry):
        for b in range(2):
            i = j * 2 + b

            @pl.when(sid + i * NS < SCHUNKS_PER_SC)
            def _(b=b, i=i):
                pltpu.make_async_copy(
                    col_hbm.at[pl.ds(0, SCHUNK)], col_v.at[b], s_col[b]).wait()
                pltpu.make_async_copy(
                    m_hbm.at[pl.ds(0, SCHUNK)], m_v.at[b], s_m[b]).wait()
                pltpu.sync_copy(m_v.at[b], sums_sh.at[col_v.at[b]], add=True)

                @pl.when(sid + (i + 2) * NS < SCHUNKS_PER_SC)
                def _():
                    start_loads(i + 2, b)

        return carry

    lax.fori_loop(0, (_SNITER + 1) // 2, body, 0)
    plsc.subcore_barrier()

    def wchunk(i, carry):
        rc = sid + i * NS

        @pl.when(rc < NRCHUNKS)
        def _():
            rbase = rc * RCHUNK
            obase = cid * N + rbase
            pltpu.sync_copy(sums_sh.at[pl.ds(rbase, RCHUNK)], zbuf_v)
            pltpu.sync_copy(zbuf_v, sums_hbm.at[pl.ds(obase, RCHUNK)])

        return carry

    lax.fori_loop(0, (NRCHUNKS + NS - 1) // NS, wchunk, 0)


def _node_body(x_ref, s_ref, p0_ref, p1_ref, p2_ref, p3_ref,
               c0_ref, c1_ref, c2_ref, c3_ref, wn2b_ref,
               wnm_ref, bnm_ref, xout_ref):
    cnt = (c0_ref[:, 0:1] + c1_ref[:, 0:1] + c2_ref[:, 0:1] + c3_ref[:, 0:1])
    agg = (p0_ref[...] + p1_ref[...] + p2_ref[...] + p3_ref[...]) / jnp.maximum(cnt, 1.0)
    h = jnp.maximum(
        s_ref[...] + jnp.dot(agg, wn2b_ref[...], preferred_element_type=jnp.float32),
        0.0)
    sh = h * jax.nn.sigmoid(h)
    xout_ref[...] = x_ref[...] + jnp.dot(
        sh, wnm_ref[...], preferred_element_type=jnp.float32) + bnm_ref[...]


def _node_stage(x, s, p0, p1, p2, p3, c0, c1, c2, c3, wn2b, wnm, bnm):
    bn = 2000
    full = lambda shape: pl.BlockSpec(shape, lambda i: (0, 0))
    return pl.pallas_call(
        _node_body,
        grid=(N // bn,),
        in_specs=[
            pl.BlockSpec((bn, H), lambda i: (i, 0)),
            pl.BlockSpec((bn, H), lambda i: (i, 0)),
            pl.BlockSpec((bn, H), lambda i: (i, 0)),
            pl.BlockSpec((bn, H), lambda i: (i, 0)),
            pl.BlockSpec((bn, H), lambda i: (i, 0)),
            pl.BlockSpec((bn, H), lambda i: (i, 0)),
            pl.BlockSpec((bn, H), lambda i: (i, 0)),
            pl.BlockSpec((bn, H), lambda i: (i, 0)),
            pl.BlockSpec((bn, H), lambda i: (i, 0)),
            pl.BlockSpec((bn, H), lambda i: (i, 0)),
            full((H, H)), full((H, H)), full((1, H)),
        ],
        out_specs=pl.BlockSpec((bn, H), lambda i: (i, 0)),
        out_shape=jax.ShapeDtypeStruct((N, H), jnp.float32),
    )(x, s, p0, p1, p2, p3, c0, c1, c2, c3, wn2b, wnm, bnm)


def kernel(x, edge_index, edge_attr, u, batch, g_n, b_n, g_e, b_e,
           We, be, Wn1, bn1, Wn2, bn2, Wnm, bnm, Wem, bem):
    row = edge_index[0]
    col = edge_index[1]
    r2 = lambda v: v.reshape(1, H)

    t, q, s = _precompute(
        x, r2(g_n), r2(b_n), We[:H], We[H:2 * H], Wn1[:H], Wn2[:H],
        r2(be), r2(bn1), r2(bn2))

    tg0, qg0, cnt0 = _gather_kernel(0)(t, q, row, col)
    tg1, qg1, cnt1 = _gather_kernel(1)(t, q, row, col)

    e_half, m0 = _edge_stage(
        0, None, tg0, qg0, edge_attr,
        r2(g_e), r2(b_e), We[2 * H:], Wn1[H:], Wem, r2(bem))
    sums0 = _scatter_kernel(0)(m0, col)

    e_out, m1 = _edge_stage(
        1, e_half, tg1, qg1, edge_attr,
        r2(g_e), r2(b_e), We[2 * H:], Wn1[H:], Wem, r2(bem))
    sums1 = _scatter_kernel(1)(m1, col)

    x_out = _node_stage(
        x, s, sums0[:N], sums0[N:], sums1[:N], sums1[N:],
        cnt0[:N], cnt0[N:], cnt1[:N], cnt1[N:], Wn2[H:], Wnm, r2(bnm))

    return (x_out, e_out)
